# baseline (speedup 1.0000x reference)
import jax
import jax.numpy as jnp
from jax import lax
from jax.experimental import pallas as pl
from jax.experimental.pallas import tpu as pltpu

B = 16
N = 32768
D = 128
BLK = 4096
NBLK = N // BLK


def _seg_onehot_t(j, starts_col, ends_col):
    rows = lax.broadcasted_iota(jnp.int32, (B, BLK), 1) + j * BLK
    mask = (rows >= starts_col) & (rows < ends_col)
    return jnp.where(mask, 1.0, 0.0)


def _body(x_ref, s_ref, o_ref, inv_ref, W2_ref, b2_ref, W1bq_ref, b1q_ref,
          W1aq_ref, gamma_ref, beta_ref, J_ref, out_ref, y_ref, sums_ref,
          c_ref):
    phase = pl.program_id(0)
    j = pl.program_id(1)
    onehot_t = _seg_onehot_t(j, s_ref[...], o_ref[...])

    @pl.when(phase == 0)
    def _phase0():
        @pl.when(j == 0)
        def _init():
            sums_ref[...] = jnp.zeros_like(sums_ref)

        x = x_ref[...]
        sums_ref[...] += lax.dot_general(
            onehot_t, x, (((1,), (0,)), ((), ())),
            preferred_element_type=jnp.float32)
        y_ref[pl.ds(j * BLK, BLK), :] = jnp.dot(
            x, W1aq_ref[...], preferred_element_type=jnp.float32)

    @pl.when(phase == 1)
    def _phase1():
        @pl.when(j == 0)
        def _pooled_mlp():
            means = sums_ref[...] * inv_ref[...]
            x_bb = jnp.maximum(
                jnp.dot(means, W2_ref[...], preferred_element_type=jnp.float32)
                + b2_ref[...], 0.0)
            c_ref[...] = jnp.dot(
                x_bb, W1bq_ref[...], preferred_element_type=jnp.float32
            ) + b1q_ref[...]

        d = y_ref[pl.ds(j * BLK, BLK), :] + lax.dot_general(
            onehot_t, c_ref[...], (((0,), (0,)), ((), ())),
            preferred_element_type=jnp.float32)
        var = jnp.dot(d * d, J_ref[...], preferred_element_type=jnp.float32)
        hn = d * lax.rsqrt(var + 1e-5) * gamma_ref[...] + beta_ref[...]
        out_ref[...] = jnp.maximum(hn, 0.0)


@jax.jit
def kernel(p, x, o, W2, b2, W1, b1, gamma, beta):
    del p
    o_col = o.reshape(B, 1)
    starts_col = jnp.concatenate(
        [jnp.zeros((1,), jnp.int32), o[:-1]]).reshape(B, 1)
    counts = jnp.concatenate([o[:1], o[1:] - o[:-1]])
    inv_counts = (1.0 / counts.astype(jnp.float32)).reshape(B, 1)

    Q = jnp.eye(D, dtype=jnp.float32) - jnp.float32(1.0 / D)
    W1aq = W1[:D] @ Q
    W1bq = W1[D:] @ Q
    b1q = (b1 @ Q).reshape(1, D)

    x_spec = pl.BlockSpec(
        (BLK, D), lambda ph, j: (jnp.where(ph == 0, j, NBLK - 1), 0))
    out_spec = pl.BlockSpec(
        (BLK, D), lambda ph, j: (jnp.where(ph == 0, 0, j), 0))
    full = lambda *shape: pl.BlockSpec(shape, lambda ph, j: (0,) * len(shape))

    return pl.pallas_call(
        _body,
        grid=(2, NBLK),
        in_specs=[x_spec, full(B, 1), full(B, 1), full(B, 1), full(D, D),
                  full(1, D), full(D, D), full(1, D), full(D, D), full(1, D),
                  full(1, D), full(D, D)],
        out_specs=out_spec,
        out_shape=jax.ShapeDtypeStruct((N, D), jnp.float32),
        scratch_shapes=[
            pltpu.VMEM((N, D), jnp.float32),
            pltpu.VMEM((B, D), jnp.float32),
            pltpu.VMEM((B, D), jnp.float32),
        ],
    )(x, starts_col, o_col, inv_counts, W2, b2.reshape(1, D), W1bq,
      b1q, W1aq, gamma.reshape(1, D), beta.reshape(1, D),
      jnp.full((D, D), 1.0 / D, jnp.float32))


# speedup vs baseline: 1.1360x; 1.1360x over previous
import jax
import jax.numpy as jnp
from jax import lax
from jax.experimental import pallas as pl
from jax.experimental.pallas import tpu as pltpu

B = 16
N = 32768
D = 128
BLK = 8192
NBLK = N // BLK


def _seg_onehot_t(j, starts_col, ends_col):
    rows = lax.broadcasted_iota(jnp.int32, (B, BLK), 1) + j * BLK
    mask = (rows >= starts_col) & (rows < ends_col)
    return jnp.where(mask, 1.0, 0.0)


def _body(x_ref, s_ref, o_ref, inv_ref, W2_ref, b2_ref, W1bq_ref, b1q_ref,
          W1aq_ref, gamma_ref, beta_ref, J_ref, out_ref, y_ref, sums_ref,
          c_ref):
    phase = pl.program_id(0)
    j = pl.program_id(1)
    onehot_t = _seg_onehot_t(j, s_ref[...], o_ref[...])

    @pl.when(phase == 0)
    def _phase0():
        @pl.when(j == 0)
        def _init():
            sums_ref[...] = jnp.zeros_like(sums_ref)

        x = x_ref[...]
        sums_ref[...] += lax.dot_general(
            onehot_t, x, (((1,), (0,)), ((), ())),
            preferred_element_type=jnp.float32)
        y_ref[pl.ds(j * BLK, BLK), :] = jnp.dot(
            x, W1aq_ref[...], preferred_element_type=jnp.float32)

    @pl.when(phase == 1)
    def _phase1():
        @pl.when(j == 0)
        def _pooled_mlp():
            means = sums_ref[...] * inv_ref[...]
            x_bb = jnp.maximum(
                jnp.dot(means, W2_ref[...], preferred_element_type=jnp.float32)
                + b2_ref[...], 0.0)
            c_ref[...] = jnp.dot(
                x_bb, W1bq_ref[...], preferred_element_type=jnp.float32
            ) + b1q_ref[...]

        d = y_ref[pl.ds(j * BLK, BLK), :] + lax.dot_general(
            onehot_t, c_ref[...], (((0,), (0,)), ((), ())),
            preferred_element_type=jnp.float32)
        var = jnp.dot(d * d, J_ref[...], preferred_element_type=jnp.float32)
        hn = d * lax.rsqrt(var + 1e-5) * gamma_ref[...] + beta_ref[...]
        out_ref[...] = jnp.maximum(hn, 0.0)


@jax.jit
def kernel(p, x, o, W2, b2, W1, b1, gamma, beta):
    del p
    o_col = o.reshape(B, 1)
    starts_col = jnp.concatenate(
        [jnp.zeros((1,), jnp.int32), o[:-1]]).reshape(B, 1)
    counts = jnp.concatenate([o[:1], o[1:] - o[:-1]])
    inv_counts = (1.0 / counts.astype(jnp.float32)).reshape(B, 1)

    Q = jnp.eye(D, dtype=jnp.float32) - jnp.float32(1.0 / D)
    W1aq = W1[:D] @ Q
    W1bq = W1[D:] @ Q
    b1q = (b1 @ Q).reshape(1, D)

    x_spec = pl.BlockSpec(
        (BLK, D), lambda ph, j: (jnp.where(ph == 0, j, NBLK - 1), 0))
    out_spec = pl.BlockSpec(
        (BLK, D), lambda ph, j: (jnp.where(ph == 0, 0, j), 0))
    full = lambda *shape: pl.BlockSpec(shape, lambda ph, j: (0,) * len(shape))

    return pl.pallas_call(
        _body,
        grid=(2, NBLK),
        in_specs=[x_spec, full(B, 1), full(B, 1), full(B, 1), full(D, D),
                  full(1, D), full(D, D), full(1, D), full(D, D), full(1, D),
                  full(1, D), full(D, D)],
        out_specs=out_spec,
        out_shape=jax.ShapeDtypeStruct((N, D), jnp.float32),
        scratch_shapes=[
            pltpu.VMEM((N, D), jnp.float32),
            pltpu.VMEM((B, D), jnp.float32),
            pltpu.VMEM((B, D), jnp.float32),
        ],
    )(x, starts_col, o_col, inv_counts, W2, b2.reshape(1, D), W1bq,
      b1q, W1aq, gamma.reshape(1, D), beta.reshape(1, D),
      jnp.full((D, D), 1.0 / D, jnp.float32))
